# baseline (speedup 1.0000x reference)
import functools

import jax
import jax.numpy as jnp
from jax import lax
from jax.experimental import pallas as pl
from jax.experimental.pallas import tpu as pltpu
from jax.experimental.pallas import tpu_sc as plsc

_NC = 2
_NS = 16
_NW = _NC * _NS
_NBUF = 2


def _pool_body(x_hbm, tab_hbm, out_hbm, idx_v, rows_bufs, acc_v, sems):
    B, L = x_hbm.shape
    D = tab_hbm.shape[1]
    bpw = B // _NW
    nd = D // 16
    wid = lax.axis_index("s") * _NC + lax.axis_index("c")
    base = wid * bpw

    pltpu.sync_copy(x_hbm.at[pl.ds(base, bpw)], idx_v)

    for b in range(_NBUF):
        pltpu.async_copy(tab_hbm.at[idx_v.at[b]], rows_bufs[b], sems[b])

    @pl.loop(0, bpw // _NBUF)
    def _(blk):
        for b in range(_NBUF):
            g = blk * _NBUF + b
            pltpu.make_async_copy(tab_hbm.at[idx_v.at[g]], rows_bufs[b],
                                  sems[b]).wait()
            rows = rows_bufs[b]

            def body(l, accs):
                return tuple(a + rows[l, pl.ds(c * 16, 16)]
                             for c, a in enumerate(accs))

            accs = lax.fori_loop(
                0, L, body,
                tuple(jnp.zeros((16,), jnp.float32) for _ in range(nd)))
            for c in range(nd):
                acc_v[g, pl.ds(c * 16, 16)] = accs[c]

            @pl.when(g + _NBUF < bpw)
            def _():
                pltpu.async_copy(tab_hbm.at[idx_v.at[g + _NBUF]], rows_bufs[b],
                                 sems[b])

    pltpu.sync_copy(acc_v, out_hbm.at[pl.ds(base, bpw)])


def _pool(x, emb_table):
    B, L = x.shape
    D = emb_table.shape[1]
    bpw = B // _NW
    mesh = plsc.VectorSubcoreMesh(core_axis_name="c", subcore_axis_name="s")

    def body(x_hbm, tab_hbm, out_hbm, idx_v, *rest):
        rows_bufs = rest[:_NBUF]
        acc_v = rest[_NBUF]
        sems = rest[_NBUF + 1:]
        _pool_body(x_hbm, tab_hbm, out_hbm, idx_v, rows_bufs, acc_v, sems)

    return pl.kernel(
        body,
        out_type=jax.ShapeDtypeStruct((B, D), jnp.float32),
        mesh=mesh,
        scratch_types=(
            [pltpu.VMEM((bpw, L), jnp.int32)]
            + [pltpu.VMEM((L, D), jnp.float32) for _ in range(_NBUF)]
            + [pltpu.VMEM((bpw, D), jnp.float32)]
            + [pltpu.SemaphoreType.DMA for _ in range(_NBUF)]
        ),
    )(x, emb_table)


def _mlp_body(s_ref, w1_ref, b1_ref, w2_ref, b2_ref, o_ref):
    hT = lax.dot_general(w1_ref[...], s_ref[...], (((0,), (1,)), ((), ())),
                         preferred_element_type=jnp.float32) + b1_ref[...]
    o_ref[...] = lax.dot_general(w2_ref[...], hT, (((0,), (0,)), ((), ())),
                                 preferred_element_type=jnp.float32) + b2_ref[...]


def _mlp(s, W1, b1, W2, b2):
    B, E = s.shape
    H = W1.shape[1]
    T = W2.shape[1]
    BT = 512
    outT = pl.pallas_call(
        _mlp_body,
        grid=(B // BT,),
        in_specs=[
            pl.BlockSpec((BT, E), lambda i: (i, 0)),
            pl.BlockSpec((E, H), lambda i: (0, 0)),
            pl.BlockSpec((H, 1), lambda i: (0, 0)),
            pl.BlockSpec((H, T), lambda i: (0, 0)),
            pl.BlockSpec((T, 1), lambda i: (0, 0)),
        ],
        out_specs=pl.BlockSpec((T, BT), lambda i: (0, i)),
        out_shape=jax.ShapeDtypeStruct((T, B), jnp.float32),
    )(s, W1, b1.reshape(H, 1), W2, b2.reshape(T, 1))
    return outT.T


def kernel(x, emb_table, W1, b1, W2, b2):
    s = _pool(x.astype(jnp.int32), emb_table)
    return _mlp(s, W1, b1, W2, b2)


# speedup vs baseline: 1.5928x; 1.5928x over previous
import functools

import jax
import jax.numpy as jnp
from jax import lax
from jax.experimental import pallas as pl
from jax.experimental.pallas import tpu as pltpu
from jax.experimental.pallas import tpu_sc as plsc

_NC = 2
_NS = 16
_NW = _NC * _NS
_NBUF = 6


def _pool_body(x_hbm, tab_hbm, out_hbm, idx_v, rows_bufs, acc_v, sems):
    B, L = x_hbm.shape
    D = tab_hbm.shape[1]
    bpw = B // _NW
    nd = D // 16
    wid = lax.axis_index("s") * _NC + lax.axis_index("c")
    base = wid * bpw

    pltpu.sync_copy(x_hbm.at[pl.ds(base, bpw)], idx_v)

    for b in range(_NBUF):
        pltpu.async_copy(tab_hbm.at[idx_v.at[b]], rows_bufs[b], sems[b])

    @pl.loop(0, bpw // _NBUF)
    def _(blk):
        for b in range(_NBUF):
            g = blk * _NBUF + b
            pltpu.make_async_copy(tab_hbm.at[idx_v.at[g]], rows_bufs[b],
                                  sems[b]).wait()
            rows = rows_bufs[b]

            def body(l, accs):
                return tuple(a + rows[l, pl.ds(c * 16, 16)]
                             for c, a in enumerate(accs))

            accs = lax.fori_loop(
                0, L, body,
                tuple(jnp.zeros((16,), jnp.float32) for _ in range(nd)))
            for c in range(nd):
                acc_v[g, pl.ds(c * 16, 16)] = accs[c]

            @pl.when(g + _NBUF < bpw)
            def _():
                pltpu.async_copy(tab_hbm.at[idx_v.at[g + _NBUF]], rows_bufs[b],
                                 sems[b])

    pltpu.sync_copy(acc_v, out_hbm.at[pl.ds(base, bpw)])


def _pool(x, emb_table):
    B, L = x.shape
    D = emb_table.shape[1]
    bpw = B // _NW
    mesh = plsc.VectorSubcoreMesh(core_axis_name="c", subcore_axis_name="s")

    def body(x_hbm, tab_hbm, out_hbm, idx_v, *rest):
        rows_bufs = rest[:_NBUF]
        acc_v = rest[_NBUF]
        sems = rest[_NBUF + 1:]
        _pool_body(x_hbm, tab_hbm, out_hbm, idx_v, rows_bufs, acc_v, sems)

    return pl.kernel(
        body,
        out_type=jax.ShapeDtypeStruct((B, D), jnp.float32),
        mesh=mesh,
        scratch_types=(
            [pltpu.VMEM((bpw, L), jnp.int32)]
            + [pltpu.VMEM((L, D), jnp.float32) for _ in range(_NBUF)]
            + [pltpu.VMEM((bpw, D), jnp.float32)]
            + [pltpu.SemaphoreType.DMA for _ in range(_NBUF)]
        ),
    )(x, emb_table)


def _mlp_body(s_ref, w1_ref, b1_ref, w2_ref, b2_ref, o_ref):
    hT = lax.dot_general(w1_ref[...], s_ref[...], (((0,), (1,)), ((), ())),
                         preferred_element_type=jnp.float32) + b1_ref[...]
    o_ref[...] = lax.dot_general(w2_ref[...], hT, (((0,), (0,)), ((), ())),
                                 preferred_element_type=jnp.float32) + b2_ref[...]


def _mlp(s, W1, b1, W2, b2):
    B, E = s.shape
    H = W1.shape[1]
    T = W2.shape[1]
    BT = 512
    outT = pl.pallas_call(
        _mlp_body,
        grid=(B // BT,),
        in_specs=[
            pl.BlockSpec((BT, E), lambda i: (i, 0)),
            pl.BlockSpec((E, H), lambda i: (0, 0)),
            pl.BlockSpec((H, 1), lambda i: (0, 0)),
            pl.BlockSpec((H, T), lambda i: (0, 0)),
            pl.BlockSpec((T, 1), lambda i: (0, 0)),
        ],
        out_specs=pl.BlockSpec((T, BT), lambda i: (0, i)),
        out_shape=jax.ShapeDtypeStruct((T, B), jnp.float32),
    )(s, W1, b1.reshape(H, 1), W2, b2.reshape(T, 1))
    return outT.T


def kernel(x, emb_table, W1, b1, W2, b2):
    s = _pool(x.astype(jnp.int32), emb_table)
    return _mlp(s, W1, b1, W2, b2)


# speedup vs baseline: 1.6052x; 1.0078x over previous
import functools

import jax
import jax.numpy as jnp
from jax import lax
from jax.experimental import pallas as pl
from jax.experimental.pallas import tpu as pltpu
from jax.experimental.pallas import tpu_sc as plsc

_NC = 2
_NS = 16
_NW = _NC * _NS
_NBUF = 8


def _pool_body(x_hbm, tab_hbm, out_hbm, idx_v, rows_bufs, acc_v, sems):
    B, L = x_hbm.shape
    D = tab_hbm.shape[1]
    bpw = B // _NW
    nd = D // 16
    wid = lax.axis_index("s") * _NC + lax.axis_index("c")
    base = wid * bpw

    pltpu.sync_copy(x_hbm.at[pl.ds(base, bpw)], idx_v)

    for b in range(_NBUF):
        pltpu.async_copy(tab_hbm.at[idx_v.at[b]], rows_bufs[b], sems[b])

    @pl.loop(0, bpw // _NBUF)
    def _(blk):
        for b in range(_NBUF):
            g = blk * _NBUF + b
            pltpu.make_async_copy(tab_hbm.at[idx_v.at[g]], rows_bufs[b],
                                  sems[b]).wait()
            rows = rows_bufs[b]

            def body(l, accs):
                return tuple(a + rows[l, pl.ds(c * 16, 16)]
                             for c, a in enumerate(accs))

            accs = lax.fori_loop(
                0, L, body,
                tuple(jnp.zeros((16,), jnp.float32) for _ in range(nd)))
            for c in range(nd):
                acc_v[g, pl.ds(c * 16, 16)] = accs[c]

            @pl.when(g + _NBUF < bpw)
            def _():
                pltpu.async_copy(tab_hbm.at[idx_v.at[g + _NBUF]], rows_bufs[b],
                                 sems[b])

    pltpu.sync_copy(acc_v, out_hbm.at[pl.ds(base, bpw)])


def _pool(x, emb_table):
    B, L = x.shape
    D = emb_table.shape[1]
    bpw = B // _NW
    mesh = plsc.VectorSubcoreMesh(core_axis_name="c", subcore_axis_name="s")

    def body(x_hbm, tab_hbm, out_hbm, idx_v, *rest):
        rows_bufs = rest[:_NBUF]
        acc_v = rest[_NBUF]
        sems = rest[_NBUF + 1:]
        _pool_body(x_hbm, tab_hbm, out_hbm, idx_v, rows_bufs, acc_v, sems)

    return pl.kernel(
        body,
        out_type=jax.ShapeDtypeStruct((B, D), jnp.float32),
        mesh=mesh,
        scratch_types=(
            [pltpu.VMEM((bpw, L), jnp.int32)]
            + [pltpu.VMEM((L, D), jnp.float32) for _ in range(_NBUF)]
            + [pltpu.VMEM((bpw, D), jnp.float32)]
            + [pltpu.SemaphoreType.DMA for _ in range(_NBUF)]
        ),
    )(x, emb_table)


def _mlp_body(s_ref, w1_ref, b1_ref, w2_ref, b2_ref, o_ref):
    hT = lax.dot_general(w1_ref[...], s_ref[...], (((0,), (1,)), ((), ())),
                         preferred_element_type=jnp.float32) + b1_ref[...]
    o_ref[...] = lax.dot_general(w2_ref[...], hT, (((0,), (0,)), ((), ())),
                                 preferred_element_type=jnp.float32) + b2_ref[...]


def _mlp(s, W1, b1, W2, b2):
    B, E = s.shape
    H = W1.shape[1]
    T = W2.shape[1]
    BT = 512
    outT = pl.pallas_call(
        _mlp_body,
        grid=(B // BT,),
        in_specs=[
            pl.BlockSpec((BT, E), lambda i: (i, 0)),
            pl.BlockSpec((E, H), lambda i: (0, 0)),
            pl.BlockSpec((H, 1), lambda i: (0, 0)),
            pl.BlockSpec((H, T), lambda i: (0, 0)),
            pl.BlockSpec((T, 1), lambda i: (0, 0)),
        ],
        out_specs=pl.BlockSpec((T, BT), lambda i: (0, i)),
        out_shape=jax.ShapeDtypeStruct((T, B), jnp.float32),
    )(s, W1, b1.reshape(H, 1), W2, b2.reshape(T, 1))
    return outT.T


def kernel(x, emb_table, W1, b1, W2, b2):
    s = _pool(x.astype(jnp.int32), emb_table)
    return _mlp(s, W1, b1, W2, b2)
